# baseline (speedup 1.0000x reference)
import functools

import jax
import jax.numpy as jnp
from jax import lax
from jax.experimental import pallas as pl
from jax.experimental.pallas import tpu as pltpu
from jax.experimental.pallas import tpu_sc as plsc

NUM_EMBED = 1000000
EMBED_DIM = 64
SHARED_DIM = 8
BATCH = 16384

NUM_CORES = 2
NUM_SUBCORES = 16
NUM_WORKERS = NUM_CORES * NUM_SUBCORES
B_PER_W = BATCH // NUM_WORKERS
L = 16
CHUNK = 32
NCHUNK = B_PER_W // CHUNK


def _body(x_hbm, w_hbm, s_hbm, out_hbm, idx_v, idxg_v, grp_v, rows_v, shared_v, sem):
    wid = lax.axis_index("s") * NUM_CORES + lax.axis_index("c")
    base = wid * B_PER_W
    pltpu.sync_copy(x_hbm.at[pl.ds(base, B_PER_W)], idx_v)
    pltpu.sync_copy(s_hbm, shared_v)

    def mk_group_idx(i, c):
        idxg_v[pl.ds(i * L, L)] = idx_v[pl.ds(i * L, L)] >> 4
        return c

    lax.fori_loop(0, B_PER_W // L, mk_group_idx, 0, unroll=8)

    io = jnp.arange(L, dtype=jnp.int32)
    col = [jnp.full((L,), c, jnp.int32) for c in range(EMBED_DIM)]
    bcast = [shared_v[c] for c in range(SHARED_DIM)]

    for k in range(NCHUNK):
        pltpu.async_copy(
            w_hbm.at[idxg_v.at[pl.ds(k * CHUNK, CHUNK)]], grp_v, sem
        ).wait()

        def pick(g, c, k=k):
            r = k * CHUNK + g * L + io
            xv = idx_v[pl.ds(k * CHUNK + g * L, L)]
            slot = g * L + io
            s_vec = (xv >> 1) & 7
            for cc in range(SHARED_DIM):
                plsc.store_scatter(rows_v, [r, col[cc]], bcast[cc])
            for cc in range(SHARED_DIM, EMBED_DIM):
                v = plsc.load_gather(
                    grp_v, [slot, s_vec, (xv & 1) * EMBED_DIM + col[cc]]
                )
                plsc.store_scatter(rows_v, [r, col[cc]], v)
            return c

        lax.fori_loop(0, CHUNK // L, pick, 0)

    pltpu.sync_copy(rows_v, out_hbm.at[pl.ds(base, B_PER_W)])


@functools.partial(
    pl.kernel,
    out_type=jax.ShapeDtypeStruct((BATCH, EMBED_DIM), jnp.float32),
    mesh=plsc.VectorSubcoreMesh(core_axis_name="c", subcore_axis_name="s"),
    scratch_types=[
        pltpu.VMEM((B_PER_W,), jnp.int32),
        pltpu.VMEM((B_PER_W,), jnp.int32),
        pltpu.VMEM((CHUNK, 8, 2 * EMBED_DIM), jnp.float32),
        pltpu.VMEM((B_PER_W, EMBED_DIM), jnp.float32),
        pltpu.VMEM((SHARED_DIM, L), jnp.float32),
        pltpu.SemaphoreType.DMA,
    ],
    compiler_params=pltpu.CompilerParams(
        skip_device_barrier=True, needs_layout_passes=False
    ),
)
def _sc_embed(x_hbm, w_hbm, s_hbm, out_hbm, idx_v, idxg_v, grp_v, rows_v, shared_v, sem):
    _body(x_hbm, w_hbm, s_hbm, out_hbm, idx_v, idxg_v, grp_v, rows_v, shared_v, sem)


def kernel(X, embed_weight, shared_embed):
    shared_bv = jnp.tile(shared_embed.reshape(SHARED_DIM, 1), (1, L))
    w3 = embed_weight.reshape(NUM_EMBED // 16, 8, 2 * EMBED_DIM)
    return _sc_embed(X.astype(jnp.int32), w3, shared_bv)


# speedup vs baseline: 2.7863x; 2.7863x over previous
import functools

import jax
import jax.numpy as jnp
from jax import lax
from jax.experimental import pallas as pl
from jax.experimental.pallas import tpu as pltpu
from jax.experimental.pallas import tpu_sc as plsc

NUM_EMBED = 1000000
EMBED_DIM = 64
SHARED_DIM = 8
BATCH = 16384

NUM_CORES = 2
NUM_SUBCORES = 16
NUM_WORKERS = NUM_CORES * NUM_SUBCORES
B_PER_W = BATCH // NUM_WORKERS
L = 16
NG = B_PER_W // L


def _body(x_hbm, w_hbm, s_hbm, out_hbm, idx_v, rows_v, shared_v, sem):
    wid = lax.axis_index("s") * NUM_CORES + lax.axis_index("c")
    base = wid * B_PER_W
    pltpu.sync_copy(x_hbm.at[pl.ds(base, B_PER_W)], idx_v)
    pltpu.sync_copy(s_hbm, shared_v)

    def drain_group():
        for _ in range(2):
            pltpu.make_async_copy(
                w_hbm.at[0], rows_v.at[pl.ds(0, 8)], sem
            ).wait()

    def fire(g, c):
        vec = idx_v[pl.ds(g * L, L)]
        for j in range(L):
            x = vec[j]
            pltpu.async_copy(w_hbm.at[x >> 3, x & 7], rows_v.at[g * L + j], sem)

        @pl.when(g >= 4)
        def _():
            drain_group()

        return c

    lax.fori_loop(0, NG, fire, 0)
    for _ in range(4):
        drain_group()

    pat = shared_v[...]
    msk = jnp.arange(L, dtype=jnp.int32) < SHARED_DIM

    def fix(i, c):
        v = rows_v[i, pl.ds(0, L)]
        rows_v[i, pl.ds(0, L)] = jnp.where(msk, pat, v)
        return c

    lax.fori_loop(0, B_PER_W, fix, 0, unroll=8)

    pltpu.sync_copy(rows_v, out_hbm.at[pl.ds(base, B_PER_W)])


@functools.partial(
    pl.kernel,
    out_type=jax.ShapeDtypeStruct((BATCH, EMBED_DIM), jnp.float32),
    mesh=plsc.VectorSubcoreMesh(core_axis_name="c", subcore_axis_name="s"),
    scratch_types=[
        pltpu.VMEM((B_PER_W,), jnp.int32),
        pltpu.VMEM((B_PER_W, EMBED_DIM), jnp.float32),
        pltpu.VMEM((L,), jnp.float32),
        pltpu.SemaphoreType.DMA,
    ],
    compiler_params=pltpu.CompilerParams(skip_device_barrier=True),
)
def _sc_embed(x_hbm, w_hbm, s_hbm, out_hbm, idx_v, rows_v, shared_v, sem):
    _body(x_hbm, w_hbm, s_hbm, out_hbm, idx_v, rows_v, shared_v, sem)


def kernel(X, embed_weight, shared_embed):
    shared16 = jnp.tile(shared_embed.reshape(-1), 2)
    w3 = embed_weight.reshape(NUM_EMBED // 8, 8, EMBED_DIM)
    return _sc_embed(X.astype(jnp.int32), w3, shared16)
